# baseline (speedup 1.0000x reference)
import functools

import jax
import jax.numpy as jnp
from jax import lax
from jax.experimental import pallas as pl
from jax.experimental.pallas import tpu as pltpu
from jax.experimental.pallas import tpu_sc as plsc

_N = 10000
_E = 320000
_D = 128
_NSUB = 16
_NW = 32
_K = 80
_EPW = _E // _NW
_NCH = _EPW // _K
_RS = 624
_R = 1000


def _each_slice(s, fn):
    fn(pl.ds(s * _RS, _RS))
    @pl.when(s == _NSUB - 1)
    def _():
        fn(pl.ds(_NSUB * _RS, _N - _NSUB * _RS))


def _make_sc_agg():
    out_type = jax.ShapeDtypeStruct((2, _N, _D), jnp.float32)
    scratch = [
        pltpu.VMEM((_NCH, _K), jnp.int32),
        pltpu.VMEM((_NCH, _K), jnp.int32),
        pltpu.VMEM((_K, _D), jnp.float32),
        pltpu.VMEM((_K, _D), jnp.float32),
        pltpu.VMEM((_K, _D), jnp.float32),
        pltpu.SemaphoreType.DMA,
        pltpu.SemaphoreType.DMA,
        pltpu.SemaphoreType.DMA,
        pltpu.SemaphoreType.DMA,
        pltpu.SemaphoreType.DMA,
        pltpu.SemaphoreType.DMA,
        pltpu.VMEM_SHARED((_N, _D), jnp.float32),
    ]

    def body(table, er, z128, out,
             src_v, dst_v, b0, b1, b2, g0, g1, g2, x0, x1, x2, acc):
        c = lax.axis_index("c")
        s = lax.axis_index("s")
        w = c * _NSUB + s
        bufs = (b0, b1, b2)
        gsem = (g0, g1, g2)
        xsem = (x0, x1, x2)

        pltpu.sync_copy(er.at[0, w], src_v)
        pltpu.sync_copy(er.at[1, w], dst_v)
        _each_slice(s, lambda sl: pltpu.sync_copy(z128.at[sl], acc.at[sl]))
        plsc.subcore_barrier()

        def gather(j, o):
            pltpu.async_copy(table.at[src_v.at[j]], bufs[o], gsem[o])

        def gwait(o):
            pltpu.make_async_copy(table.at[src_v.at[0]], bufs[o],
                                  gsem[o]).wait()

        def scat(j, o):
            pltpu.async_copy(bufs[o], acc.at[dst_v.at[j]], xsem[o],
                             add=True)

        def swait(o):
            pltpu.make_async_copy(bufs[o], acc.at[dst_v.at[0]],
                                  xsem[o]).wait()

        gather(0, 0)
        gather(1, 1)
        gwait(0)
        scat(0, 0)
        gather(2, 2)

        def steady(i, carry):
            for o_idx in range(3):
                j = 3 * i + 1 + o_idx
                o = (1 + o_idx) % 3
                nslot = (o + 2) % 3
                gwait(o)
                scat(j, o)
                @pl.when(j + 2 < _NCH)
                def _():
                    swait(nslot)
                    gather(j + 2, nslot)
            return carry

        lax.fori_loop(0, (_NCH - 2) // 3, steady, 0)
        gwait((_NCH - 1) % 3)
        scat(_NCH - 1, (_NCH - 1) % 3)
        for o in range(3):
            swait(o)
        plsc.subcore_barrier()
        _each_slice(s, lambda sl: pltpu.sync_copy(acc.at[sl],
                                                  out.at[c, sl]))

    mesh = plsc.VectorSubcoreMesh(core_axis_name="c", subcore_axis_name="s")
    return pl.kernel(
        body, out_type=out_type, mesh=mesh, scratch_types=scratch,
        compiler_params=pltpu.CompilerParams(use_tc_tiling_on_sc=False))


def _make_sc_deg():
    out_type = jax.ShapeDtypeStruct((2, _N, 16), jnp.float32)
    scratch = [
        pltpu.VMEM((_NCH, _K), jnp.int32),
        pltpu.VMEM((_K, 16), jnp.float32),
        pltpu.VMEM_SHARED((_N, 16), jnp.float32),
    ]

    def body(er, z16, degp, dst_v, ones_v, dega):
        c = lax.axis_index("c")
        s = lax.axis_index("s")
        w = c * _NSUB + s
        pltpu.sync_copy(er.at[1, w], dst_v)
        _each_slice(s, lambda sl: pltpu.sync_copy(z16.at[sl], dega.at[sl]))
        for i in range(_K):
            ones_v[i, :] = jnp.ones((16,), jnp.float32)
        plsc.subcore_barrier()

        def chunk(j, carry):
            pltpu.sync_copy(ones_v, dega.at[dst_v.at[j]], add=True)
            return carry

        lax.fori_loop(0, _NCH, chunk, 0)
        plsc.subcore_barrier()
        _each_slice(s, lambda sl: pltpu.sync_copy(dega.at[sl],
                                                  degp.at[c, sl]))

    mesh = plsc.VectorSubcoreMesh(core_axis_name="c", subcore_axis_name="s")
    return pl.kernel(
        body, out_type=out_type, mesh=mesh, scratch_types=scratch,
        compiler_params=pltpu.CompilerParams(use_tc_tiling_on_sc=False))


@functools.cache
def _get_agg():
    return _make_sc_agg()


@functools.cache
def _get_deg():
    return _make_sc_deg()


def _dot(a, b):
    return jnp.dot(a, b, preferred_element_type=jnp.float32)


def _encoder_body(mref, lref, tref, wm, bm, wl, bl, wt, bt, o0, o1, o2):
    m = bm[...]
    for cch in range(8):
        m = m + _dot(mref[:, cch, :], wm[cch])
    o0[...] = jnp.maximum(m, 0.0)
    o1[...] = jnp.maximum(_dot(lref[...], wl[...]) + bl[...], 0.0)
    o2[...] = jnp.maximum(_dot(tref[...], wt[...]) + bt[...], 0.0)


def _encoder_call(metric, logx, tracex, wm8, bm, wl, bl, wt, bt):
    grid = (_N // _R,)
    row = lambda i: (i, 0)
    row3 = lambda i: (i, 0, 0)
    full = lambda i: (0, 0)
    full3 = lambda i: (0, 0, 0)
    return pl.pallas_call(
        _encoder_body,
        grid=grid,
        in_specs=[
            pl.BlockSpec((_R, 8, 64), row3),
            pl.BlockSpec((_R, 64), row),
            pl.BlockSpec((_R, 64), row),
            pl.BlockSpec((8, 64, _D), full3),
            pl.BlockSpec((1, _D), full),
            pl.BlockSpec((64, _D), full),
            pl.BlockSpec((1, _D), full),
            pl.BlockSpec((64, _D), full),
            pl.BlockSpec((1, _D), full),
        ],
        out_specs=[pl.BlockSpec((_R, _D), row)] * 3,
        out_shape=[jax.ShapeDtypeStruct((_N, _D), jnp.float32)] * 3,
    )(metric, logx, tracex, wm8, bm, wl, bl, wt, bt)


def _layer_body(h, sp, degref, ws, wn, b, o):
    deg = degref[0, :, 0:1] + degref[1, :, 0:1]
    inv = 1.0 / jnp.maximum(deg, 1.0)
    neigh = (sp[0] + sp[1]) * inv
    o[...] = jnp.maximum(
        _dot(h[...], ws[...]) + _dot(neigh, wn[...]) + b[...], 0.0)


def _layer_call(h, sp, degp, ws, wn, bgl):
    grid = (_N // _R,)
    row = lambda i: (i, 0)
    prow = lambda i: (0, i, 0)
    full2 = lambda i: (0, 0)
    return pl.pallas_call(
        _layer_body,
        grid=grid,
        in_specs=[
            pl.BlockSpec((_R, _D), row),
            pl.BlockSpec((2, _R, _D), prow),
            pl.BlockSpec((2, _R, 16), prow),
            pl.BlockSpec((_D, _D), full2),
            pl.BlockSpec((_D, _D), full2),
            pl.BlockSpec((1, _D), full2),
        ],
        out_specs=pl.BlockSpec((_R, _D), row),
        out_shape=jax.ShapeDtypeStruct((_N, _D), jnp.float32),
    )(h, sp, degp, ws, wn, bgl)


def _fusion_body(h0, h1, h2, s0, s1, s2, degref, ws, wn, b,
                 aw, av, wc1, bc1, wc2, bc2, wv1, bv1, wv2, bv2,
                 eref, rootref, fref, typeref, sums):
    i = pl.program_id(0)

    @pl.when(i == 0)
    def _():
        sums[...] = jnp.zeros_like(sums)

    deg = degref[0, :, 0:1] + degref[1, :, 0:1]
    inv = 1.0 / jnp.maximum(deg, 1.0)
    hs = []
    for mi, (h, sp) in enumerate(((h0, s0), (h1, s1), (h2, s2))):
        neigh = (sp[0] + sp[1]) * inv
        hs.append(jnp.maximum(
            _dot(h[...], ws[mi]) + _dot(neigh, wn[mi]) + b[mi, :], 0.0))
    m, l, t = hs
    sums[0:1, :] = sums[0:1, :] + jnp.sum(m, axis=0, keepdims=True)
    sums[1:2, :] = sums[1:2, :] + jnp.sum(l, axis=0, keepdims=True)
    sums[2:3, :] = sums[2:3, :] + jnp.sum(t, axis=0, keepdims=True)

    def score(h):
        return jnp.sum(jnp.tanh(_dot(h, aw[...])) * av[...],
                       axis=1, keepdims=True)

    scm, scl, sct = score(m), score(l), score(t)
    mx = jnp.maximum(jnp.maximum(scm, scl), sct)
    em = jnp.exp(scm - mx)
    el = jnp.exp(scl - mx)
    et = jnp.exp(sct - mx)
    den = em + el + et
    e = (em * m + el * l + et * t) / den
    eref[...] = e
    ea = jnp.maximum(_dot(e, wv1[...]) + bv1[...], 0.0)
    rootref[...] = _dot(ea, wv2[...]) + bv2[...]

    @pl.when(i == (_N // _R) - 1)
    def _():
        fs = sums[0:3, :] * (1.0 / _N)
        fsc = jnp.sum(jnp.tanh(_dot(fs, aw[...])) * av[...],
                      axis=1, keepdims=True)
        fmx = jnp.max(fsc)
        fe = jnp.exp(fsc - fmx)
        fa = fe / jnp.sum(fe)
        f = jnp.sum(fa * fs, axis=0, keepdims=True)
        fref[...] = f
        fact = jnp.maximum(_dot(f, wc1[...]) + bc1[...], 0.0)
        typeref[...] = _dot(fact, wc2[...]) + bc2[...]


def _fusion_call(h0, h1, h2, s0, s1, s2, degp, ws, wn, bgl,
                 aw, av, wc1, bc1, wc2p, bc2p, wv1, bv1, wv2p, bv2p):
    grid = (_N // _R,)
    row = lambda i: (i, 0)
    prow = lambda i: (0, i, 0)
    full = lambda i: (0, 0)
    full3 = lambda i: (0, 0, 0)
    return pl.pallas_call(
        _fusion_body,
        grid=grid,
        in_specs=[
            pl.BlockSpec((_R, _D), row),
            pl.BlockSpec((_R, _D), row),
            pl.BlockSpec((_R, _D), row),
            pl.BlockSpec((2, _R, _D), prow),
            pl.BlockSpec((2, _R, _D), prow),
            pl.BlockSpec((2, _R, _D), prow),
            pl.BlockSpec((2, _R, 16), prow),
            pl.BlockSpec((3, _D, _D), full3),
            pl.BlockSpec((3, _D, _D), full3),
            pl.BlockSpec((3, _D), full),
            pl.BlockSpec((_D, 64), full),
            pl.BlockSpec((1, 64), full),
            pl.BlockSpec((_D, _D), full),
            pl.BlockSpec((1, _D), full),
            pl.BlockSpec((_D, _D), full),
            pl.BlockSpec((1, _D), full),
            pl.BlockSpec((_D, _D), full),
            pl.BlockSpec((1, _D), full),
            pl.BlockSpec((_D, _D), full),
            pl.BlockSpec((1, _D), full),
        ],
        out_specs=[
            pl.BlockSpec((_R, _D), row),
            pl.BlockSpec((_R, _D), row),
            pl.BlockSpec((1, _D), full),
            pl.BlockSpec((1, _D), full),
        ],
        out_shape=[
            jax.ShapeDtypeStruct((_N, _D), jnp.float32),
            jax.ShapeDtypeStruct((_N, _D), jnp.float32),
            jax.ShapeDtypeStruct((1, _D), jnp.float32),
            jax.ShapeDtypeStruct((1, _D), jnp.float32),
        ],
        scratch_shapes=[pltpu.VMEM((8, _D), jnp.float32)],
    )(h0, h1, h2, s0, s1, s2, degp, ws, wn, bgl,
      aw, av, wc1, bc1, wc2p, bc2p, wv1, bv1, wv2p, bv2p)


def kernel(metric, log, trace, edge_index, W_metric, b_metric, W_log, b_log,
           W_trace, b_trace, Wg_self, Wg_neigh, bg, att_W, att_v,
           Wc1, bc1, Wc2, bc2, Wv1, bv1, Wv2, bv2):
    n = metric.shape[0]
    er = edge_index.reshape(2, _NW, _NCH, _K)
    z128 = jnp.zeros((n, _D), jnp.float32)
    z16 = jnp.zeros((n, 16), jnp.float32)
    degp = _get_deg()(er, z16)

    h0, h1, h2 = _encoder_call(
        metric, log, trace,
        W_metric.reshape(8, 64, _D), b_metric.reshape(1, -1),
        W_log, b_log.reshape(1, -1),
        W_trace, b_trace.reshape(1, -1))

    agg = _get_agg()
    hs = [h0, h1, h2]
    sp1 = [agg(h, er, z128) for h in hs]
    hs = [_layer_call(hs[m], sp1[m], degp,
                      Wg_self[m, 0], Wg_neigh[m, 0], bg[m, 0].reshape(1, -1))
          for m in range(3)]
    sp2 = [agg(h, er, z128) for h in hs]
    h0, h1, h2 = hs
    s0, s1, s2 = sp2

    wc2p = jnp.zeros((_D, _D), jnp.float32).at[:, :Wc2.shape[1]].set(Wc2)
    bc2p = jnp.zeros((1, _D), jnp.float32).at[0, :bc2.shape[0]].set(bc2)
    wv2p = jnp.zeros((_D, _D), jnp.float32).at[:, :Wv2.shape[1]].set(Wv2)
    bv2p = jnp.zeros((1, _D), jnp.float32).at[0, :bv2.shape[0]].set(bv2)

    e, root128, f1, type1 = _fusion_call(
        h0, h1, h2, s0, s1, s2, degp,
        Wg_self[:, 1], Wg_neigh[:, 1], bg[:, 1],
        att_W, att_v.reshape(1, -1),
        Wc1, bc1.reshape(1, -1), wc2p, bc2p,
        Wv1, bv1.reshape(1, -1), wv2p, bv2p)

    root_logit = root128[:, :1]
    type_logit = type1[0, :Wc2.shape[1]]
    return (root_logit, type_logit, f1[0], e)


# speedup vs baseline: 1.0728x; 1.0728x over previous
import functools

import jax
import jax.numpy as jnp
from jax import lax
from jax.experimental import pallas as pl
from jax.experimental.pallas import tpu as pltpu
from jax.experimental.pallas import tpu_sc as plsc

_N = 10000
_E = 320000
_D = 128
_NSUB = 16
_NW = 32
_K = 80
_EPW = _E // _NW
_NCH = _EPW // _K
_RS = 624
_R = 1000


def _each_slice(s, fn):
    fn(pl.ds(s * _RS, _RS))
    @pl.when(s == _NSUB - 1)
    def _():
        fn(pl.ds(_NSUB * _RS, _N - _NSUB * _RS))


def _make_sc_agg(nt):
    out_type = [jax.ShapeDtypeStruct((2, _N, _D), jnp.float32)
                for _ in range(nt)]
    scratch = [
        pltpu.VMEM((_NCH, _K), jnp.int32),
        pltpu.VMEM((_NCH, _K), jnp.int32),
        pltpu.VMEM((_K, _D), jnp.float32),
        pltpu.VMEM((_K, _D), jnp.float32),
        pltpu.VMEM((_K, _D), jnp.float32),
        pltpu.SemaphoreType.DMA,
        pltpu.SemaphoreType.DMA,
        pltpu.SemaphoreType.DMA,
        pltpu.SemaphoreType.DMA,
        pltpu.SemaphoreType.DMA,
        pltpu.SemaphoreType.DMA,
        pltpu.VMEM_SHARED((_N, _D), jnp.float32),
    ]

    def body(*refs):
        tables = refs[:nt]
        er, z128 = refs[nt:nt + 2]
        outs = refs[nt + 2:2 * nt + 2]
        (src_v, dst_v, b0, b1, b2,
         g0, g1, g2, x0, x1, x2, acc) = refs[2 * nt + 2:]
        c = lax.axis_index("c")
        s = lax.axis_index("s")
        w = c * _NSUB + s
        bufs = (b0, b1, b2)
        gsem = (g0, g1, g2)
        xsem = (x0, x1, x2)

        pltpu.sync_copy(er.at[0, w], src_v)
        pltpu.sync_copy(er.at[1, w], dst_v)
        _each_slice(s, lambda sl: pltpu.sync_copy(z128.at[sl], acc.at[sl]))
        plsc.subcore_barrier()

        for t in range(nt):
            table, out = tables[t], outs[t]

            def gather(j, o):
                pltpu.async_copy(table.at[src_v.at[j]], bufs[o], gsem[o])

            def gwait(o):
                pltpu.make_async_copy(table.at[src_v.at[0]], bufs[o],
                                      gsem[o]).wait()

            def scat(j, o):
                pltpu.async_copy(bufs[o], acc.at[dst_v.at[j]], xsem[o],
                                 add=True)

            def swait(o):
                pltpu.make_async_copy(bufs[o], acc.at[dst_v.at[0]],
                                      xsem[o]).wait()

            gather(0, 0)
            gather(1, 1)
            gwait(0)
            scat(0, 0)
            gather(2, 2)

            def steady(i, carry):
                for o_idx in range(3):
                    j = 3 * i + 1 + o_idx
                    o = (1 + o_idx) % 3
                    nslot = (o + 2) % 3
                    gwait(o)
                    scat(j, o)
                    @pl.when(j + 2 < _NCH)
                    def _():
                        swait(nslot)
                        gather(j + 2, nslot)
                return carry

            lax.fori_loop(0, (_NCH - 2) // 3, steady, 0)
            gwait((_NCH - 1) % 3)
            scat(_NCH - 1, (_NCH - 1) % 3)
            for o in range(3):
                swait(o)
            plsc.subcore_barrier()

            def writeout(sl):
                pltpu.sync_copy(acc.at[sl], out.at[c, sl])
                if t < nt - 1:
                    pltpu.sync_copy(z128.at[sl], acc.at[sl])

            _each_slice(s, writeout)
            if t < nt - 1:
                plsc.subcore_barrier()

    mesh = plsc.VectorSubcoreMesh(core_axis_name="c", subcore_axis_name="s")
    return pl.kernel(
        body, out_type=out_type, mesh=mesh, scratch_types=scratch,
        compiler_params=pltpu.CompilerParams(use_tc_tiling_on_sc=False))


def _make_sc_deg():
    out_type = jax.ShapeDtypeStruct((2, _N, 16), jnp.float32)
    scratch = [
        pltpu.VMEM((_NCH, _K), jnp.int32),
        pltpu.VMEM((_K, 16), jnp.float32),
        pltpu.VMEM_SHARED((_N, 16), jnp.float32),
    ]

    def body(er, z16, degp, dst_v, ones_v, dega):
        c = lax.axis_index("c")
        s = lax.axis_index("s")
        w = c * _NSUB + s
        pltpu.sync_copy(er.at[1, w], dst_v)
        _each_slice(s, lambda sl: pltpu.sync_copy(z16.at[sl], dega.at[sl]))
        for i in range(_K):
            ones_v[i, :] = jnp.ones((16,), jnp.float32)
        plsc.subcore_barrier()

        def chunk(j, carry):
            pltpu.sync_copy(ones_v, dega.at[dst_v.at[j]], add=True)
            return carry

        lax.fori_loop(0, _NCH, chunk, 0)
        plsc.subcore_barrier()
        _each_slice(s, lambda sl: pltpu.sync_copy(dega.at[sl],
                                                  degp.at[c, sl]))

    mesh = plsc.VectorSubcoreMesh(core_axis_name="c", subcore_axis_name="s")
    return pl.kernel(
        body, out_type=out_type, mesh=mesh, scratch_types=scratch,
        compiler_params=pltpu.CompilerParams(use_tc_tiling_on_sc=False))


@functools.cache
def _get_agg(nt):
    return _make_sc_agg(nt)


@functools.cache
def _get_deg():
    return _make_sc_deg()


def _dot(a, b):
    return jnp.dot(a, b, preferred_element_type=jnp.float32)


def _enc_lt_body(lref, tref, wl, bl, wt, bt, o1, o2):
    o1[...] = jnp.maximum(_dot(lref[...], wl[...]) + bl[...], 0.0)
    o2[...] = jnp.maximum(_dot(tref[...], wt[...]) + bt[...], 0.0)


def _enc_lt_call(logx, tracex, wl, bl, wt, bt):
    grid = (_N // _R,)
    row = lambda i: (i, 0)
    full = lambda i: (0, 0)
    return pl.pallas_call(
        _enc_lt_body,
        grid=grid,
        in_specs=[
            pl.BlockSpec((_R, 64), row),
            pl.BlockSpec((_R, 64), row),
            pl.BlockSpec((64, _D), full),
            pl.BlockSpec((1, _D), full),
            pl.BlockSpec((64, _D), full),
            pl.BlockSpec((1, _D), full),
        ],
        out_specs=[pl.BlockSpec((_R, _D), row)] * 2,
        out_shape=[jax.ShapeDtypeStruct((_N, _D), jnp.float32)] * 2,
    )(logx, tracex, wl, bl, wt, bt)


def _enc_m_body(mref, wm, bm, o0):
    o0[...] = jnp.maximum(_dot(mref[...], wm[...]) + bm[...], 0.0)


def _enc_m_call(metric2, wm, bm):
    grid = (_N // _R,)
    row = lambda i: (i, 0)
    full = lambda i: (0, 0)
    return pl.pallas_call(
        _enc_m_body,
        grid=grid,
        in_specs=[
            pl.BlockSpec((_R, 512), row),
            pl.BlockSpec((512, _D), full),
            pl.BlockSpec((1, _D), full),
        ],
        out_specs=pl.BlockSpec((_R, _D), row),
        out_shape=jax.ShapeDtypeStruct((_N, _D), jnp.float32),
    )(metric2, wm, bm)


def _upd_body(degref, ws, wn, b, *hso):
    nm = len(hso) // 3
    hs, sps, os = hso[:nm], hso[nm:2 * nm], hso[2 * nm:]
    deg = degref[0, :, 0:1] + degref[1, :, 0:1]
    inv = 1.0 / jnp.maximum(deg, 1.0)
    for mi in range(nm):
        neigh = (sps[mi][0] + sps[mi][1]) * inv
        os[mi][...] = jnp.maximum(
            _dot(hs[mi][...], ws[mi]) + _dot(neigh, wn[mi]) + b[mi, :], 0.0)


def _upd_call(hs, sps, degp, ws, wn, bgl):
    nm = len(hs)
    grid = (_N // _R,)
    row = lambda i: (i, 0)
    prow = lambda i: (0, i, 0)
    full2 = lambda i: (0, 0)
    full3 = lambda i: (0, 0, 0)
    out = pl.pallas_call(
        _upd_body,
        grid=grid,
        in_specs=(
            [pl.BlockSpec((2, _R, 16), prow),
             pl.BlockSpec((nm, _D, _D), full3),
             pl.BlockSpec((nm, _D, _D), full3),
             pl.BlockSpec((nm, _D), full2)]
            + [pl.BlockSpec((_R, _D), row)] * nm
            + [pl.BlockSpec((2, _R, _D), prow)] * nm
        ),
        out_specs=[pl.BlockSpec((_R, _D), row)] * nm,
        out_shape=[jax.ShapeDtypeStruct((_N, _D), jnp.float32)] * nm,
    )(degp, ws, wn, bgl, *hs, *sps)
    return out


def _fusion_body(h0, h1, h2, s0, s1, s2, degref, ws, wn, b,
                 aw, av, wc1, bc1, wc2, bc2, wv1, bv1, wv2, bv2,
                 eref, rootref, fref, typeref, sums):
    i = pl.program_id(0)

    @pl.when(i == 0)
    def _():
        sums[...] = jnp.zeros_like(sums)

    deg = degref[0, :, 0:1] + degref[1, :, 0:1]
    inv = 1.0 / jnp.maximum(deg, 1.0)
    hs = []
    for mi, (h, sp) in enumerate(((h0, s0), (h1, s1), (h2, s2))):
        neigh = (sp[0] + sp[1]) * inv
        hs.append(jnp.maximum(
            _dot(h[...], ws[mi]) + _dot(neigh, wn[mi]) + b[mi, :], 0.0))
    m, l, t = hs
    sums[0:1, :] = sums[0:1, :] + jnp.sum(m, axis=0, keepdims=True)
    sums[1:2, :] = sums[1:2, :] + jnp.sum(l, axis=0, keepdims=True)
    sums[2:3, :] = sums[2:3, :] + jnp.sum(t, axis=0, keepdims=True)

    def score(h):
        return jnp.sum(jnp.tanh(_dot(h, aw[...])) * av[...],
                       axis=1, keepdims=True)

    scm, scl, sct = score(m), score(l), score(t)
    mx = jnp.maximum(jnp.maximum(scm, scl), sct)
    em = jnp.exp(scm - mx)
    el = jnp.exp(scl - mx)
    et = jnp.exp(sct - mx)
    den = em + el + et
    e = (em * m + el * l + et * t) / den
    eref[...] = e
    ea = jnp.maximum(_dot(e, wv1[...]) + bv1[...], 0.0)
    rootref[...] = _dot(ea, wv2[...]) + bv2[...]

    @pl.when(i == (_N // _R) - 1)
    def _():
        fs = sums[0:3, :] * (1.0 / _N)
        fsc = jnp.sum(jnp.tanh(_dot(fs, aw[...])) * av[...],
                      axis=1, keepdims=True)
        fmx = jnp.max(fsc)
        fe = jnp.exp(fsc - fmx)
        fa = fe / jnp.sum(fe)
        f = jnp.sum(fa * fs, axis=0, keepdims=True)
        fref[...] = f
        fact = jnp.maximum(_dot(f, wc1[...]) + bc1[...], 0.0)
        typeref[...] = _dot(fact, wc2[...]) + bc2[...]


def _fusion_call(h0, h1, h2, s0, s1, s2, degp, ws, wn, bgl,
                 aw, av, wc1, bc1, wc2p, bc2p, wv1, bv1, wv2p, bv2p):
    grid = (_N // _R,)
    row = lambda i: (i, 0)
    prow = lambda i: (0, i, 0)
    full = lambda i: (0, 0)
    full3 = lambda i: (0, 0, 0)
    return pl.pallas_call(
        _fusion_body,
        grid=grid,
        in_specs=[
            pl.BlockSpec((_R, _D), row),
            pl.BlockSpec((_R, _D), row),
            pl.BlockSpec((_R, _D), row),
            pl.BlockSpec((2, _R, _D), prow),
            pl.BlockSpec((2, _R, _D), prow),
            pl.BlockSpec((2, _R, _D), prow),
            pl.BlockSpec((2, _R, 16), prow),
            pl.BlockSpec((3, _D, _D), full3),
            pl.BlockSpec((3, _D, _D), full3),
            pl.BlockSpec((3, _D), full),
            pl.BlockSpec((_D, 64), full),
            pl.BlockSpec((1, 64), full),
            pl.BlockSpec((_D, _D), full),
            pl.BlockSpec((1, _D), full),
            pl.BlockSpec((_D, _D), full),
            pl.BlockSpec((1, _D), full),
            pl.BlockSpec((_D, _D), full),
            pl.BlockSpec((1, _D), full),
            pl.BlockSpec((_D, _D), full),
            pl.BlockSpec((1, _D), full),
        ],
        out_specs=[
            pl.BlockSpec((_R, _D), row),
            pl.BlockSpec((_R, _D), row),
            pl.BlockSpec((1, _D), full),
            pl.BlockSpec((1, _D), full),
        ],
        out_shape=[
            jax.ShapeDtypeStruct((_N, _D), jnp.float32),
            jax.ShapeDtypeStruct((_N, _D), jnp.float32),
            jax.ShapeDtypeStruct((1, _D), jnp.float32),
            jax.ShapeDtypeStruct((1, _D), jnp.float32),
        ],
        scratch_shapes=[pltpu.VMEM((8, _D), jnp.float32)],
    )(h0, h1, h2, s0, s1, s2, degp, ws, wn, bgl,
      aw, av, wc1, bc1, wc2p, bc2p, wv1, bv1, wv2p, bv2p)


def kernel(metric, log, trace, edge_index, W_metric, b_metric, W_log, b_log,
           W_trace, b_trace, Wg_self, Wg_neigh, bg, att_W, att_v,
           Wc1, bc1, Wc2, bc2, Wv1, bv1, Wv2, bv2):
    n = metric.shape[0]
    er = edge_index.reshape(2, _NW, _NCH, _K)
    z128 = jnp.zeros((n, _D), jnp.float32)
    z16 = jnp.zeros((n, 16), jnp.float32)
    degp = _get_deg()(er, z16)

    h1, h2 = _enc_lt_call(log, trace, W_log, b_log.reshape(1, -1),
                          W_trace, b_trace.reshape(1, -1))
    s1a, s2a = _get_agg(2)(h1, h2, er, z128)
    h0 = _enc_m_call(metric.reshape(n, -1), W_metric,
                     b_metric.reshape(1, -1))
    (s0a,) = _get_agg(1)(h0, er, z128)
    h1, h2 = _upd_call([h1, h2], [s1a, s2a], degp,
                       Wg_self[1:, 0], Wg_neigh[1:, 0], bg[1:, 0])
    s1, s2 = _get_agg(2)(h1, h2, er, z128)
    (h0,) = _upd_call([h0], [s0a], degp,
                      Wg_self[0:1, 0], Wg_neigh[0:1, 0], bg[0:1, 0])
    (s0,) = _get_agg(1)(h0, er, z128)

    wc2p = jnp.zeros((_D, _D), jnp.float32).at[:, :Wc2.shape[1]].set(Wc2)
    bc2p = jnp.zeros((1, _D), jnp.float32).at[0, :bc2.shape[0]].set(bc2)
    wv2p = jnp.zeros((_D, _D), jnp.float32).at[:, :Wv2.shape[1]].set(Wv2)
    bv2p = jnp.zeros((1, _D), jnp.float32).at[0, :bv2.shape[0]].set(bv2)

    e, root128, f1, type1 = _fusion_call(
        h0, h1, h2, s0, s1, s2, degp,
        Wg_self[:, 1], Wg_neigh[:, 1], bg[:, 1],
        att_W, att_v.reshape(1, -1),
        Wc1, bc1.reshape(1, -1), wc2p, bc2p,
        Wv1, bv1.reshape(1, -1), wv2p, bv2p)

    root_logit = root128[:, :1]
    type_logit = type1[0, :Wc2.shape[1]]
    return (root_logit, type_logit, f1[0], e)


# speedup vs baseline: 1.0807x; 1.0073x over previous
import functools

import jax
import jax.numpy as jnp
from jax import lax
from jax.experimental import pallas as pl
from jax.experimental.pallas import tpu as pltpu
from jax.experimental.pallas import tpu_sc as plsc

_N = 10000
_E = 320000
_D = 128
_NSUB = 16
_NW = 32
_K = 80
_EPW = _E // _NW
_NCH = _EPW // _K
_RS = 624
_R = 1000


def _each_slice(s, fn):
    fn(pl.ds(s * _RS, _RS))
    @pl.when(s == _NSUB - 1)
    def _():
        fn(pl.ds(_NSUB * _RS, _N - _NSUB * _RS))


def _make_sc_agg(nt):
    out_type = [jax.ShapeDtypeStruct((2, _N, _D), jnp.float32)
                for _ in range(nt)]
    scratch = [
        pltpu.VMEM((_NCH, _K), jnp.int32),
        pltpu.VMEM((_NCH, _K), jnp.int32),
        pltpu.VMEM((_K, _D), jnp.float32),
        pltpu.VMEM((_K, _D), jnp.float32),
        pltpu.VMEM((_K, _D), jnp.float32),
        pltpu.SemaphoreType.DMA,
        pltpu.SemaphoreType.DMA,
        pltpu.SemaphoreType.DMA,
        pltpu.SemaphoreType.DMA,
        pltpu.SemaphoreType.DMA,
        pltpu.SemaphoreType.DMA,
        pltpu.VMEM_SHARED((_N, _D), jnp.float32),
    ]

    def body(*refs):
        tables = refs[:nt]
        er, z128 = refs[nt:nt + 2]
        outs = refs[nt + 2:2 * nt + 2]
        (src_v, dst_v, b0, b1, b2,
         g0, g1, g2, x0, x1, x2, acc) = refs[2 * nt + 2:]
        c = lax.axis_index("c")
        s = lax.axis_index("s")
        w = c * _NSUB + s
        bufs = (b0, b1, b2)
        gsem = (g0, g1, g2)
        xsem = (x0, x1, x2)

        pltpu.sync_copy(er.at[0, w], src_v)
        pltpu.sync_copy(er.at[1, w], dst_v)
        _each_slice(s, lambda sl: pltpu.sync_copy(z128.at[sl], acc.at[sl]))
        plsc.subcore_barrier()

        for t in range(nt):
            table, out = tables[t], outs[t]

            def gather(j, o):
                pltpu.async_copy(table.at[src_v.at[j]], bufs[o], gsem[o])

            def gwait(o):
                pltpu.make_async_copy(table.at[src_v.at[0]], bufs[o],
                                      gsem[o]).wait()

            def scat(j, o):
                pltpu.async_copy(bufs[o], acc.at[dst_v.at[j]], xsem[o],
                                 add=True)

            def swait(o):
                pltpu.make_async_copy(bufs[o], acc.at[dst_v.at[0]],
                                      xsem[o]).wait()

            gather(0, 0)
            gather(1, 1)
            gwait(0)
            scat(0, 0)
            gather(2, 2)

            def steady(i, carry):
                for o_idx in range(3):
                    j = 3 * i + 1 + o_idx
                    o = (1 + o_idx) % 3
                    nslot = (o + 2) % 3
                    gwait(o)
                    scat(j, o)
                    @pl.when(j + 2 < _NCH)
                    def _():
                        swait(nslot)
                        gather(j + 2, nslot)
                return carry

            lax.fori_loop(0, (_NCH - 2) // 3, steady, 0)
            gwait((_NCH - 1) % 3)
            scat(_NCH - 1, (_NCH - 1) % 3)
            for o in range(3):
                swait(o)
            plsc.subcore_barrier()

            def writeout(sl):
                pltpu.sync_copy(acc.at[sl], out.at[c, sl])
                if t < nt - 1:
                    pltpu.sync_copy(z128.at[sl], acc.at[sl])

            _each_slice(s, writeout)
            if t < nt - 1:
                plsc.subcore_barrier()

    mesh = plsc.VectorSubcoreMesh(core_axis_name="c", subcore_axis_name="s")
    return pl.kernel(
        body, out_type=out_type, mesh=mesh, scratch_types=scratch,
        compiler_params=pltpu.CompilerParams(use_tc_tiling_on_sc=False))


def _make_sc_deg():
    out_type = jax.ShapeDtypeStruct((2, _N, 16), jnp.float32)
    scratch = [
        pltpu.VMEM((_NCH, _K), jnp.int32),
        pltpu.VMEM((_K, 16), jnp.float32),
        pltpu.VMEM_SHARED((_N, 16), jnp.float32),
    ]

    def body(er, z16, degp, dst_v, ones_v, dega):
        c = lax.axis_index("c")
        s = lax.axis_index("s")
        w = c * _NSUB + s
        pltpu.sync_copy(er.at[1, w], dst_v)
        _each_slice(s, lambda sl: pltpu.sync_copy(z16.at[sl], dega.at[sl]))
        for i in range(_K):
            ones_v[i, :] = jnp.ones((16,), jnp.float32)
        plsc.subcore_barrier()

        def chunk(j, carry):
            pltpu.sync_copy(ones_v, dega.at[dst_v.at[j]], add=True)
            return carry

        lax.fori_loop(0, _NCH, chunk, 0)
        plsc.subcore_barrier()
        _each_slice(s, lambda sl: pltpu.sync_copy(dega.at[sl],
                                                  degp.at[c, sl]))

    mesh = plsc.VectorSubcoreMesh(core_axis_name="c", subcore_axis_name="s")
    return pl.kernel(
        body, out_type=out_type, mesh=mesh, scratch_types=scratch,
        compiler_params=pltpu.CompilerParams(use_tc_tiling_on_sc=False))


@functools.cache
def _get_agg(nt):
    return _make_sc_agg(nt)


@functools.cache
def _get_deg():
    return _make_sc_deg()


def _dot(a, b):
    return jnp.dot(a, b, preferred_element_type=jnp.float32)


def _enc_lt_body(lref, tref, wl, bl, wt, bt, o1, o2):
    o1[...] = jnp.maximum(_dot(lref[...], wl[...]) + bl[...], 0.0)
    o2[...] = jnp.maximum(_dot(tref[...], wt[...]) + bt[...], 0.0)


def _enc_lt_call(logx, tracex, wl, bl, wt, bt):
    grid = (_N // _R,)
    row = lambda i: (i, 0)
    full = lambda i: (0, 0)
    return pl.pallas_call(
        _enc_lt_body,
        grid=grid,
        in_specs=[
            pl.BlockSpec((_R, 64), row),
            pl.BlockSpec((_R, 64), row),
            pl.BlockSpec((64, _D), full),
            pl.BlockSpec((1, _D), full),
            pl.BlockSpec((64, _D), full),
            pl.BlockSpec((1, _D), full),
        ],
        out_specs=[pl.BlockSpec((_R, _D), row)] * 2,
        out_shape=[jax.ShapeDtypeStruct((_N, _D), jnp.float32)] * 2,
    )(logx, tracex, wl, bl, wt, bt)


def _enc_m_body(mref, wm, bm, o0):
    o0[...] = jnp.maximum(_dot(mref[...], wm[...]) + bm[...], 0.0)


def _enc_m_call(metric2, wm, bm):
    grid = (_N // _R,)
    row = lambda i: (i, 0)
    full = lambda i: (0, 0)
    return pl.pallas_call(
        _enc_m_body,
        grid=grid,
        in_specs=[
            pl.BlockSpec((_R, 512), row),
            pl.BlockSpec((512, _D), full),
            pl.BlockSpec((1, _D), full),
        ],
        out_specs=pl.BlockSpec((_R, _D), row),
        out_shape=jax.ShapeDtypeStruct((_N, _D), jnp.float32),
    )(metric2, wm, bm)


def _upd_body(degref, ws, wn, b, *hso):
    nm = len(hso) // 3
    hs, sps, os = hso[:nm], hso[nm:2 * nm], hso[2 * nm:]
    deg = degref[0, :, 0:1] + degref[1, :, 0:1]
    inv = 1.0 / jnp.maximum(deg, 1.0)
    for mi in range(nm):
        neigh = (sps[mi][0] + sps[mi][1]) * inv
        os[mi][...] = jnp.maximum(
            _dot(hs[mi][...], ws[mi]) + _dot(neigh, wn[mi]) + b[mi, :], 0.0)


def _upd_call(hs, sps, degp, ws, wn, bgl):
    nm = len(hs)
    grid = (_N // _R,)
    row = lambda i: (i, 0)
    prow = lambda i: (0, i, 0)
    full2 = lambda i: (0, 0)
    full3 = lambda i: (0, 0, 0)
    out = pl.pallas_call(
        _upd_body,
        grid=grid,
        in_specs=(
            [pl.BlockSpec((2, _R, 16), prow),
             pl.BlockSpec((nm, _D, _D), full3),
             pl.BlockSpec((nm, _D, _D), full3),
             pl.BlockSpec((nm, _D), full2)]
            + [pl.BlockSpec((_R, _D), row)] * nm
            + [pl.BlockSpec((2, _R, _D), prow)] * nm
        ),
        out_specs=[pl.BlockSpec((_R, _D), row)] * nm,
        out_shape=[jax.ShapeDtypeStruct((_N, _D), jnp.float32)] * nm,
    )(degp, ws, wn, bgl, *hs, *sps)
    return out


def _fusion_body(h0, h1, h2, s0, degref, ws, wn, b,
                 aw, av, wc1, bc1, wc2, bc2, wv1, bv1, wv2, bv2,
                 eref, rootref, fref, typeref, sums):
    i = pl.program_id(0)

    @pl.when(i == 0)
    def _():
        sums[...] = jnp.zeros_like(sums)

    deg = degref[0, :, 0:1] + degref[1, :, 0:1]
    inv = 1.0 / jnp.maximum(deg, 1.0)
    neigh = (s0[0] + s0[1]) * inv
    m = jnp.maximum(_dot(h0[...], ws[...]) + _dot(neigh, wn[...]) + b[...],
                    0.0)
    l, t = h1[...], h2[...]
    sums[0:1, :] = sums[0:1, :] + jnp.sum(m, axis=0, keepdims=True)
    sums[1:2, :] = sums[1:2, :] + jnp.sum(l, axis=0, keepdims=True)
    sums[2:3, :] = sums[2:3, :] + jnp.sum(t, axis=0, keepdims=True)

    def score(h):
        return jnp.sum(jnp.tanh(_dot(h, aw[...])) * av[...],
                       axis=1, keepdims=True)

    scm, scl, sct = score(m), score(l), score(t)
    mx = jnp.maximum(jnp.maximum(scm, scl), sct)
    em = jnp.exp(scm - mx)
    el = jnp.exp(scl - mx)
    et = jnp.exp(sct - mx)
    den = em + el + et
    e = (em * m + el * l + et * t) / den
    eref[...] = e
    ea = jnp.maximum(_dot(e, wv1[...]) + bv1[...], 0.0)
    rootref[...] = _dot(ea, wv2[...]) + bv2[...]

    @pl.when(i == (_N // _R) - 1)
    def _():
        fs = sums[0:3, :] * (1.0 / _N)
        fsc = jnp.sum(jnp.tanh(_dot(fs, aw[...])) * av[...],
                      axis=1, keepdims=True)
        fmx = jnp.max(fsc)
        fe = jnp.exp(fsc - fmx)
        fa = fe / jnp.sum(fe)
        f = jnp.sum(fa * fs, axis=0, keepdims=True)
        fref[...] = f
        fact = jnp.maximum(_dot(f, wc1[...]) + bc1[...], 0.0)
        typeref[...] = _dot(fact, wc2[...]) + bc2[...]


def _fusion_call(h0, h1, h2, s0, degp, ws, wn, bgl,
                 aw, av, wc1, bc1, wc2p, bc2p, wv1, bv1, wv2p, bv2p):
    grid = (_N // _R,)
    row = lambda i: (i, 0)
    prow = lambda i: (0, i, 0)
    full = lambda i: (0, 0)
    return pl.pallas_call(
        _fusion_body,
        grid=grid,
        in_specs=[
            pl.BlockSpec((_R, _D), row),
            pl.BlockSpec((_R, _D), row),
            pl.BlockSpec((_R, _D), row),
            pl.BlockSpec((2, _R, _D), prow),
            pl.BlockSpec((2, _R, 16), prow),
            pl.BlockSpec((_D, _D), full),
            pl.BlockSpec((_D, _D), full),
            pl.BlockSpec((1, _D), full),
            pl.BlockSpec((_D, 64), full),
            pl.BlockSpec((1, 64), full),
            pl.BlockSpec((_D, _D), full),
            pl.BlockSpec((1, _D), full),
            pl.BlockSpec((_D, _D), full),
            pl.BlockSpec((1, _D), full),
            pl.BlockSpec((_D, _D), full),
            pl.BlockSpec((1, _D), full),
            pl.BlockSpec((_D, _D), full),
            pl.BlockSpec((1, _D), full),
        ],
        out_specs=[
            pl.BlockSpec((_R, _D), row),
            pl.BlockSpec((_R, _D), row),
            pl.BlockSpec((1, _D), full),
            pl.BlockSpec((1, _D), full),
        ],
        out_shape=[
            jax.ShapeDtypeStruct((_N, _D), jnp.float32),
            jax.ShapeDtypeStruct((_N, _D), jnp.float32),
            jax.ShapeDtypeStruct((1, _D), jnp.float32),
            jax.ShapeDtypeStruct((1, _D), jnp.float32),
        ],
        scratch_shapes=[pltpu.VMEM((8, _D), jnp.float32)],
    )(h0, h1, h2, s0, degp, ws, wn, bgl,
      aw, av, wc1, bc1, wc2p, bc2p, wv1, bv1, wv2p, bv2p)


def kernel(metric, log, trace, edge_index, W_metric, b_metric, W_log, b_log,
           W_trace, b_trace, Wg_self, Wg_neigh, bg, att_W, att_v,
           Wc1, bc1, Wc2, bc2, Wv1, bv1, Wv2, bv2):
    n = metric.shape[0]
    er = edge_index.reshape(2, _NW, _NCH, _K)
    z128 = jnp.zeros((n, _D), jnp.float32)
    z16 = jnp.zeros((n, 16), jnp.float32)
    degp = _get_deg()(er, z16)

    h1, h2 = _enc_lt_call(log, trace, W_log, b_log.reshape(1, -1),
                          W_trace, b_trace.reshape(1, -1))
    s1a, s2a = _get_agg(2)(h1, h2, er, z128)
    metric_b, _ = lax.optimization_barrier((metric, h1))
    h0 = _enc_m_call(metric_b.reshape(n, -1), W_metric,
                     b_metric.reshape(1, -1))
    (s0a,) = _get_agg(1)(h0, er, z128)
    h1, h2 = _upd_call([h1, h2], [s1a, s2a], degp,
                       Wg_self[1:, 0], Wg_neigh[1:, 0], bg[1:, 0])
    s1, s2 = _get_agg(2)(h1, h2, er, z128)
    (h0,) = _upd_call([h0], [s0a], degp,
                      Wg_self[0:1, 0], Wg_neigh[0:1, 0], bg[0:1, 0])
    (s0,) = _get_agg(1)(h0, er, z128)
    h1, h2 = _upd_call([h1, h2], [s1, s2], degp,
                       Wg_self[1:, 1], Wg_neigh[1:, 1], bg[1:, 1])

    wc2p = jnp.zeros((_D, _D), jnp.float32).at[:, :Wc2.shape[1]].set(Wc2)
    bc2p = jnp.zeros((1, _D), jnp.float32).at[0, :bc2.shape[0]].set(bc2)
    wv2p = jnp.zeros((_D, _D), jnp.float32).at[:, :Wv2.shape[1]].set(Wv2)
    bv2p = jnp.zeros((1, _D), jnp.float32).at[0, :bv2.shape[0]].set(bv2)

    e, root128, f1, type1 = _fusion_call(
        h0, h1, h2, s0, degp,
        Wg_self[0, 1], Wg_neigh[0, 1], bg[0, 1].reshape(1, -1),
        att_W, att_v.reshape(1, -1),
        Wc1, bc1.reshape(1, -1), wc2p, bc2p,
        Wv1, bv1.reshape(1, -1), wv2p, bv2p)

    root_logit = root128[:, :1]
    type_logit = type1[0, :Wc2.shape[1]]
    return (root_logit, type_logit, f1[0], e)
